# baseline (speedup 1.0000x reference)
import functools

import jax
import jax.numpy as jnp
from jax import lax
from jax.experimental import pallas as pl
from jax.experimental.pallas import tpu as pltpu
from jax.experimental.pallas import tpu_sc as plsc

N, H, I, E = 2048, 1024, 2048, 8
T = 128
NT_R = (2 * N) // T + E
P = NT_R * T
TS = 256
NT = N // TS
NW = 32
TPW = N // NW


def _plan_body(x_ref, gw_ref, bias_ref, sw1_ref, sw2_ref, sw3_ref,
               pos0_ref, pos1_ref, w0_ref, w1_ref, te_ref, xbf_ref,
               sw1b_ref, sw2b_ref, sw3b_ref):
    x = x_ref[...]
    gw = gw_ref[...]
    bias = bias_ref[...]
    xbf_ref[...] = x.astype(jnp.bfloat16)
    sw1b_ref[...] = sw1_ref[...].astype(jnp.bfloat16)
    sw2b_ref[...] = sw2_ref[...].astype(jnp.bfloat16)
    sw3b_ref[...] = sw3_ref[...].astype(jnp.bfloat16)
    logits = lax.dot_general(x, gw, (((1,), (1,)), ((), ())),
                             preferred_element_type=jnp.float32)
    biased = logits + bias
    eiota = lax.broadcasted_iota(jnp.int32, (N, E), 1)
    m0 = jnp.max(biased, axis=1, keepdims=True)
    e0 = jnp.min(jnp.where(biased >= m0, eiota, E), axis=1, keepdims=True)
    oh0 = eiota == e0
    biased2 = jnp.where(oh0, -jnp.inf, biased)
    m1 = jnp.max(biased2, axis=1, keepdims=True)
    e1 = jnp.min(jnp.where(biased2 >= m1, eiota, E), axis=1, keepdims=True)
    oh1 = eiota == e1
    g0 = jnp.sum(jnp.where(oh0, logits, 0.0), axis=1, keepdims=True)
    g1 = jnp.sum(jnp.where(oh1, logits, 0.0), axis=1, keepdims=True)
    mm = jnp.maximum(g0, g1)
    z0 = jnp.exp(g0 - mm)
    z1 = jnp.exp(g1 - mm)
    w0_ref[...] = z0 / (z0 + z1)
    w1_ref[...] = z1 / (z0 + z1)
    both = oh0.astype(jnp.float32) + oh1.astype(jnp.float32)
    r = lax.broadcasted_iota(jnp.int32, (N, N), 0)
    c = lax.broadcasted_iota(jnp.int32, (N, N), 1)
    ltri = (c < r).astype(jnp.bfloat16)
    cex = lax.dot_general(ltri, both.astype(jnp.bfloat16),
                          (((1,), (0,)), ((), ())),
                          preferred_element_type=jnp.float32)
    counts = jnp.sum(both, axis=0, keepdims=True)
    tiles = jnp.floor((counts + (T - 1)) * (1.0 / T))
    er = lax.broadcasted_iota(jnp.int32, (E, E), 0)
    ec = lax.broadcasted_iota(jnp.int32, (E, E), 1)
    strict = (er < ec).astype(jnp.float32)
    toff = lax.dot_general(tiles, strict, (((1,), (0,)), ((), ())),
                           preferred_element_type=jnp.float32)
    off_row = toff * float(T)
    pos0_ref[...] = jnp.sum(jnp.where(oh0, off_row + cex, 0.0), axis=1,
                            keepdims=True).astype(jnp.int32)
    pos1_ref[...] = jnp.sum(jnp.where(oh1, off_row + cex, 0.0), axis=1,
                            keepdims=True).astype(jnp.int32)
    bnd = (toff + tiles).astype(jnp.int32)
    s_row = lax.broadcasted_iota(jnp.int32, (NT_R, E), 0)
    te = jnp.sum((bnd <= s_row).astype(jnp.int32), axis=1, keepdims=True)
    te_ref[...] = te


def _plan(x, gate_w, expert_bias, sw1, sw2, sw3):
    return pl.pallas_call(
        _plan_body,
        out_shape=(
            jax.ShapeDtypeStruct((N, 1), jnp.int32),
            jax.ShapeDtypeStruct((N, 1), jnp.int32),
            jax.ShapeDtypeStruct((N, 1), jnp.float32),
            jax.ShapeDtypeStruct((N, 1), jnp.float32),
            jax.ShapeDtypeStruct((NT_R, 1), jnp.int32),
            jax.ShapeDtypeStruct((N, H), jnp.bfloat16),
            jax.ShapeDtypeStruct(sw1.shape, jnp.bfloat16),
            jax.ShapeDtypeStruct(sw2.shape, jnp.bfloat16),
            jax.ShapeDtypeStruct(sw3.shape, jnp.bfloat16),
        ),
    )(x, gate_w, expert_bias.reshape(1, E), sw1, sw2, sw3)


@functools.cache
def _dispatch_kernel():
    @functools.partial(
        pl.kernel,
        out_type=jax.ShapeDtypeStruct((P, H), jnp.float32),
        mesh=plsc.VectorSubcoreMesh(core_axis_name="c", subcore_axis_name="s"),
        scratch_types=[
            pltpu.VMEM((TPW,), jnp.int32),
            pltpu.VMEM((TPW,), jnp.int32),
            pltpu.VMEM((TPW, H), jnp.float32),
            pltpu.SemaphoreType.DMA,
        ],
    )
    def _dispatch(x_hbm, pos0_hbm, pos1_hbm, xs_hbm, idx0_v, idx1_v, rows_v,
                  sem):
        wid = lax.axis_index("s") * 2 + lax.axis_index("c")
        base = wid * TPW
        pltpu.sync_copy(pos0_hbm.at[pl.ds(base, TPW)], idx0_v)
        pltpu.sync_copy(pos1_hbm.at[pl.ds(base, TPW)], idx1_v)
        pltpu.sync_copy(x_hbm.at[pl.ds(base, TPW)], rows_v)
        pltpu.async_copy(rows_v, xs_hbm.at[idx0_v], sem).wait()
        pltpu.async_copy(rows_v, xs_hbm.at[idx1_v], sem).wait()

    return _dispatch


def _gemm_body(te_ref, x_ref, w1_ref, w3_ref, w2_ref, o_ref):
    i = pl.program_id(0)
    j = pl.program_id(1)
    te = te_ref[i]

    @pl.when(te < E)
    def _():
        x = x_ref[...]
        w1b = w1_ref[0]
        w3b = w3_ref[0]
        w2b = w2_ref[0]
        h1 = lax.dot_general(x, w1b, (((1,), (1,)), ((), ())),
                             preferred_element_type=jnp.float32)
        h3 = lax.dot_general(x, w3b, (((1,), (1,)), ((), ())),
                             preferred_element_type=jnp.float32)
        g = h1 * jax.nn.sigmoid(h1) * h3
        contrib = lax.dot_general(g, w2b, (((1,), (1,)), ((), ())),
                                  preferred_element_type=jnp.float32)

        @pl.when(j == 0)
        def _():
            o_ref[...] = contrib

        @pl.when(j == 1)
        def _():
            o_ref[...] += contrib


def _wsel(te, i):
    return jnp.minimum(te[i], E - 1)


def _routed_gemm(te, xs, w1b, w2b, w3b):
    grid_spec = pltpu.PrefetchScalarGridSpec(
        num_scalar_prefetch=1,
        grid=(NT_R, 2),
        in_specs=[
            pl.BlockSpec((T, H), lambda i, j, te: (i, 0)),
            pl.BlockSpec((1, I // 2, H), lambda i, j, te: (_wsel(te, i), j, 0)),
            pl.BlockSpec((1, I // 2, H), lambda i, j, te: (_wsel(te, i), j, 0)),
            pl.BlockSpec((1, H, I // 2), lambda i, j, te: (_wsel(te, i), 0, j)),
        ],
        out_specs=pl.BlockSpec((T, H), lambda i, j, te: (i, 0)),
    )
    return pl.pallas_call(
        _gemm_body,
        grid_spec=grid_spec,
        out_shape=jax.ShapeDtypeStruct((P, H), jnp.float32),
        compiler_params=pltpu.CompilerParams(
            dimension_semantics=("arbitrary", "arbitrary")),
    )(te, xs, w1b, w3b, w2b)


@functools.cache
def _combine_kernel():
    @functools.partial(
        pl.kernel,
        out_type=(jax.ShapeDtypeStruct((N, H), jnp.float32),
                  jax.ShapeDtypeStruct((N, H), jnp.float32)),
        mesh=plsc.VectorSubcoreMesh(core_axis_name="c", subcore_axis_name="s"),
        scratch_types=[
            pltpu.VMEM((TPW,), jnp.int32),
            pltpu.VMEM((TPW, H), jnp.float32),
            pltpu.SemaphoreType.DMA,
        ],
    )
    def _combine(ys_hbm, pos0_hbm, pos1_hbm, zs0_hbm, zs1_hbm, idx_v, rows_v,
                 sem):
        wid = lax.axis_index("s") * 2 + lax.axis_index("c")
        base = wid * TPW
        pltpu.sync_copy(pos0_hbm.at[pl.ds(base, TPW)], idx_v)
        pltpu.async_copy(ys_hbm.at[idx_v], rows_v, sem).wait()
        pltpu.sync_copy(rows_v, zs0_hbm.at[pl.ds(base, TPW)])
        pltpu.sync_copy(pos1_hbm.at[pl.ds(base, TPW)], idx_v)
        pltpu.async_copy(ys_hbm.at[idx_v], rows_v, sem).wait()
        pltpu.sync_copy(rows_v, zs1_hbm.at[pl.ds(base, TPW)])

    return _combine


def _shared_body(x_ref, w1_ref, w3_ref, w2_ref, o_ref):
    x = x_ref[...]
    h1 = lax.dot_general(x, w1_ref[...], (((1,), (1,)), ((), ())),
                         preferred_element_type=jnp.float32)
    h3 = lax.dot_general(x, w3_ref[...], (((1,), (1,)), ((), ())),
                         preferred_element_type=jnp.float32)
    g = (h1 * jax.nn.sigmoid(h1) * h3).astype(jnp.bfloat16)
    o_ref[...] = lax.dot_general(g, w2_ref[...], (((1,), (1,)), ((), ())),
                                 preferred_element_type=jnp.float32)


def _shared_gemm(xbf, sw1b, sw2b, sw3b):
    return pl.pallas_call(
        _shared_body,
        grid=(NT,),
        in_specs=[
            pl.BlockSpec((TS, H), lambda i: (i, 0)),
            pl.BlockSpec((I, H), lambda i: (0, 0)),
            pl.BlockSpec((I, H), lambda i: (0, 0)),
            pl.BlockSpec((H, I), lambda i: (0, 0)),
        ],
        out_specs=pl.BlockSpec((TS, H), lambda i: (i, 0)),
        out_shape=jax.ShapeDtypeStruct((N, H), jnp.float32),
        compiler_params=pltpu.CompilerParams(
            dimension_semantics=("arbitrary",)),
    )(xbf, sw1b, sw3b, sw2b)


def _fin_body(sh_ref, z0_ref, z1_ref, w0_ref, w1_ref, o_ref):
    o_ref[...] = (sh_ref[...] + w0_ref[...] * z0_ref[...]
                  + w1_ref[...] * z1_ref[...])


def _final(sh, zs0, zs1, rw0, rw1):
    return pl.pallas_call(
        _fin_body,
        grid=(NT,),
        in_specs=[
            pl.BlockSpec((TS, H), lambda i: (i, 0)),
            pl.BlockSpec((TS, H), lambda i: (i, 0)),
            pl.BlockSpec((TS, H), lambda i: (i, 0)),
            pl.BlockSpec((TS, 1), lambda i: (i, 0)),
            pl.BlockSpec((TS, 1), lambda i: (i, 0)),
        ],
        out_specs=pl.BlockSpec((TS, H), lambda i: (i, 0)),
        out_shape=jax.ShapeDtypeStruct((N, H), jnp.float32),
        compiler_params=pltpu.CompilerParams(
            dimension_semantics=("parallel",)),
    )(sh, zs0, zs1, rw0, rw1)


def kernel(hidden_states, gate_w, expert_bias, w1, w2, w3, sw1, sw2, sw3):
    x = hidden_states.reshape(N, H)
    (pos0, pos1, rw0, rw1, te, xbf,
     sw1b, sw2b, sw3b) = _plan(x, gate_w, expert_bias, sw1, sw2, sw3)
    pos0f = pos0.reshape(N)
    pos1f = pos1.reshape(N)
    xs = _dispatch_kernel()(x, pos0f, pos1f)
    ys = _routed_gemm(te.reshape(NT_R), xs, w1, w2, w3)
    sh = _shared_gemm(xbf, sw1b, sw2b, sw3b)
    zs0, zs1 = _combine_kernel()(ys, pos0f, pos1f)
    out = _final(sh, zs0, zs1, rw0, rw1)
    return out.reshape(1, N, H)


# speedup vs baseline: 1.7980x; 1.7980x over previous
import functools

import jax
import jax.numpy as jnp
from jax import lax
from jax.experimental import pallas as pl
from jax.experimental.pallas import tpu as pltpu
from jax.experimental.pallas import tpu_sc as plsc

N, H, I, E = 2048, 1024, 2048, 8
T = 256
NT_R = (2 * N) // T + E
P = NT_R * T
TS = 256
NT = N // TS
NW = 32
TPW = N // NW


def _plan_body(x_ref, gw_ref, bias_ref, sw1_ref, sw2_ref, sw3_ref,
               pos0_ref, pos1_ref, w0_ref, w1_ref, te_ref, xbf_ref,
               sw1b_ref, sw2b_ref, sw3b_ref):
    x = x_ref[...]
    gw = gw_ref[...]
    bias = bias_ref[...]
    xbf_ref[...] = x.astype(jnp.bfloat16)
    sw1b_ref[...] = sw1_ref[...].astype(jnp.bfloat16)
    sw2b_ref[...] = sw2_ref[...].astype(jnp.bfloat16)
    sw3b_ref[...] = sw3_ref[...].astype(jnp.bfloat16)
    logits = lax.dot_general(x, gw, (((1,), (1,)), ((), ())),
                             preferred_element_type=jnp.float32)
    biased = logits + bias
    eiota = lax.broadcasted_iota(jnp.int32, (N, E), 1)
    m0 = jnp.max(biased, axis=1, keepdims=True)
    e0 = jnp.min(jnp.where(biased >= m0, eiota, E), axis=1, keepdims=True)
    oh0 = eiota == e0
    biased2 = jnp.where(oh0, -jnp.inf, biased)
    m1 = jnp.max(biased2, axis=1, keepdims=True)
    e1 = jnp.min(jnp.where(biased2 >= m1, eiota, E), axis=1, keepdims=True)
    oh1 = eiota == e1
    g0 = jnp.sum(jnp.where(oh0, logits, 0.0), axis=1, keepdims=True)
    g1 = jnp.sum(jnp.where(oh1, logits, 0.0), axis=1, keepdims=True)
    mm = jnp.maximum(g0, g1)
    z0 = jnp.exp(g0 - mm)
    z1 = jnp.exp(g1 - mm)
    w0_ref[...] = z0 / (z0 + z1)
    w1_ref[...] = z1 / (z0 + z1)
    both = oh0.astype(jnp.float32) + oh1.astype(jnp.float32)
    r = lax.broadcasted_iota(jnp.int32, (N, N), 0)
    c = lax.broadcasted_iota(jnp.int32, (N, N), 1)
    ltri = (c < r).astype(jnp.bfloat16)
    cex = lax.dot_general(ltri, both.astype(jnp.bfloat16),
                          (((1,), (0,)), ((), ())),
                          preferred_element_type=jnp.float32)
    counts = jnp.sum(both, axis=0, keepdims=True)
    tiles = jnp.floor((counts + (T - 1)) * (1.0 / T))
    er = lax.broadcasted_iota(jnp.int32, (E, E), 0)
    ec = lax.broadcasted_iota(jnp.int32, (E, E), 1)
    strict = (er < ec).astype(jnp.float32)
    toff = lax.dot_general(tiles, strict, (((1,), (0,)), ((), ())),
                           preferred_element_type=jnp.float32)
    off_row = toff * float(T)
    pos0_ref[...] = jnp.sum(jnp.where(oh0, off_row + cex, 0.0), axis=1,
                            keepdims=True).astype(jnp.int32)
    pos1_ref[...] = jnp.sum(jnp.where(oh1, off_row + cex, 0.0), axis=1,
                            keepdims=True).astype(jnp.int32)
    bnd = (toff + tiles).astype(jnp.int32)
    s_row = lax.broadcasted_iota(jnp.int32, (NT_R, E), 0)
    te = jnp.sum((bnd <= s_row).astype(jnp.int32), axis=1, keepdims=True)
    te_ref[...] = te


def _plan(x, gate_w, expert_bias, sw1, sw2, sw3):
    return pl.pallas_call(
        _plan_body,
        out_shape=(
            jax.ShapeDtypeStruct((N, 1), jnp.int32),
            jax.ShapeDtypeStruct((N, 1), jnp.int32),
            jax.ShapeDtypeStruct((N, 1), jnp.float32),
            jax.ShapeDtypeStruct((N, 1), jnp.float32),
            jax.ShapeDtypeStruct((NT_R, 1), jnp.int32),
            jax.ShapeDtypeStruct((N, H), jnp.bfloat16),
            jax.ShapeDtypeStruct(sw1.shape, jnp.bfloat16),
            jax.ShapeDtypeStruct(sw2.shape, jnp.bfloat16),
            jax.ShapeDtypeStruct(sw3.shape, jnp.bfloat16),
        ),
    )(x, gate_w, expert_bias.reshape(1, E), sw1, sw2, sw3)


@functools.cache
def _dispatch_kernel():
    @functools.partial(
        pl.kernel,
        out_type=jax.ShapeDtypeStruct((P, H), jnp.float32),
        mesh=plsc.VectorSubcoreMesh(core_axis_name="c", subcore_axis_name="s"),
        scratch_types=[
            pltpu.VMEM((TPW,), jnp.int32),
            pltpu.VMEM((TPW,), jnp.int32),
            pltpu.VMEM((TPW, H), jnp.float32),
            pltpu.SemaphoreType.DMA,
        ],
    )
    def _dispatch(x_hbm, pos0_hbm, pos1_hbm, xs_hbm, idx0_v, idx1_v, rows_v,
                  sem):
        wid = lax.axis_index("s") * 2 + lax.axis_index("c")
        base = wid * TPW
        pltpu.sync_copy(pos0_hbm.at[pl.ds(base, TPW)], idx0_v)
        pltpu.sync_copy(pos1_hbm.at[pl.ds(base, TPW)], idx1_v)
        pltpu.sync_copy(x_hbm.at[pl.ds(base, TPW)], rows_v)
        pltpu.async_copy(rows_v, xs_hbm.at[idx0_v], sem).wait()
        pltpu.async_copy(rows_v, xs_hbm.at[idx1_v], sem).wait()

    return _dispatch


def _gemm_body(te_ref, x_ref, w1_ref, w3_ref, w2_ref, o_ref):
    i = pl.program_id(0)
    te = te_ref[i]

    @pl.when(te < E)
    def _():
        x = x_ref[...]
        w1b = w1_ref[0]
        w3b = w3_ref[0]
        w2b = w2_ref[0]
        h1 = lax.dot_general(x, w1b, (((1,), (1,)), ((), ())),
                             preferred_element_type=jnp.float32)
        h3 = lax.dot_general(x, w3b, (((1,), (1,)), ((), ())),
                             preferred_element_type=jnp.float32)
        g = h1 * jax.nn.sigmoid(h1) * h3
        o_ref[...] = lax.dot_general(g, w2b, (((1,), (1,)), ((), ())),
                                     preferred_element_type=jnp.float32)


def _wsel(te, i):
    return jnp.minimum(te[i], E - 1)


def _routed_gemm(te, xs, w1b, w2b, w3b):
    grid_spec = pltpu.PrefetchScalarGridSpec(
        num_scalar_prefetch=1,
        grid=(NT_R,),
        in_specs=[
            pl.BlockSpec((T, H), lambda i, te: (i, 0)),
            pl.BlockSpec((1, I, H), lambda i, te: (_wsel(te, i), 0, 0)),
            pl.BlockSpec((1, I, H), lambda i, te: (_wsel(te, i), 0, 0)),
            pl.BlockSpec((1, H, I), lambda i, te: (_wsel(te, i), 0, 0)),
        ],
        out_specs=pl.BlockSpec((T, H), lambda i, te: (i, 0)),
    )
    return pl.pallas_call(
        _gemm_body,
        grid_spec=grid_spec,
        out_shape=jax.ShapeDtypeStruct((P, H), jnp.float32),
        compiler_params=pltpu.CompilerParams(
            dimension_semantics=("arbitrary",)),
    )(te, xs, w1b, w3b, w2b)


@functools.cache
def _combine_kernel():
    @functools.partial(
        pl.kernel,
        out_type=(jax.ShapeDtypeStruct((N, H), jnp.float32),
                  jax.ShapeDtypeStruct((N, H), jnp.float32)),
        mesh=plsc.VectorSubcoreMesh(core_axis_name="c", subcore_axis_name="s"),
        scratch_types=[
            pltpu.VMEM((TPW,), jnp.int32),
            pltpu.VMEM((TPW, H), jnp.float32),
            pltpu.SemaphoreType.DMA,
        ],
    )
    def _combine(ys_hbm, pos0_hbm, pos1_hbm, zs0_hbm, zs1_hbm, idx_v, rows_v,
                 sem):
        wid = lax.axis_index("s") * 2 + lax.axis_index("c")
        base = wid * TPW
        pltpu.sync_copy(pos0_hbm.at[pl.ds(base, TPW)], idx_v)
        pltpu.async_copy(ys_hbm.at[idx_v], rows_v, sem).wait()
        pltpu.sync_copy(rows_v, zs0_hbm.at[pl.ds(base, TPW)])
        pltpu.sync_copy(pos1_hbm.at[pl.ds(base, TPW)], idx_v)
        pltpu.async_copy(ys_hbm.at[idx_v], rows_v, sem).wait()
        pltpu.sync_copy(rows_v, zs1_hbm.at[pl.ds(base, TPW)])

    return _combine


def _shared_body(x_ref, w1_ref, w3_ref, w2_ref, o_ref):
    x = x_ref[...]
    h1 = lax.dot_general(x, w1_ref[...], (((1,), (1,)), ((), ())),
                         preferred_element_type=jnp.float32)
    h3 = lax.dot_general(x, w3_ref[...], (((1,), (1,)), ((), ())),
                         preferred_element_type=jnp.float32)
    g = (h1 * jax.nn.sigmoid(h1) * h3).astype(jnp.bfloat16)
    o_ref[...] = lax.dot_general(g, w2_ref[...], (((1,), (1,)), ((), ())),
                                 preferred_element_type=jnp.float32)


def _shared_gemm(xbf, sw1b, sw2b, sw3b):
    return pl.pallas_call(
        _shared_body,
        grid=(NT,),
        in_specs=[
            pl.BlockSpec((TS, H), lambda i: (i, 0)),
            pl.BlockSpec((I, H), lambda i: (0, 0)),
            pl.BlockSpec((I, H), lambda i: (0, 0)),
            pl.BlockSpec((H, I), lambda i: (0, 0)),
        ],
        out_specs=pl.BlockSpec((TS, H), lambda i: (i, 0)),
        out_shape=jax.ShapeDtypeStruct((N, H), jnp.float32),
        compiler_params=pltpu.CompilerParams(
            dimension_semantics=("arbitrary",)),
    )(xbf, sw1b, sw3b, sw2b)


def _fin_body(sh_ref, z0_ref, z1_ref, w0_ref, w1_ref, o_ref):
    o_ref[...] = (sh_ref[...] + w0_ref[...] * z0_ref[...]
                  + w1_ref[...] * z1_ref[...])


def _final(sh, zs0, zs1, rw0, rw1):
    return pl.pallas_call(
        _fin_body,
        grid=(NT,),
        in_specs=[
            pl.BlockSpec((TS, H), lambda i: (i, 0)),
            pl.BlockSpec((TS, H), lambda i: (i, 0)),
            pl.BlockSpec((TS, H), lambda i: (i, 0)),
            pl.BlockSpec((TS, 1), lambda i: (i, 0)),
            pl.BlockSpec((TS, 1), lambda i: (i, 0)),
        ],
        out_specs=pl.BlockSpec((TS, H), lambda i: (i, 0)),
        out_shape=jax.ShapeDtypeStruct((N, H), jnp.float32),
        compiler_params=pltpu.CompilerParams(
            dimension_semantics=("parallel",)),
    )(sh, zs0, zs1, rw0, rw1)


def kernel(hidden_states, gate_w, expert_bias, w1, w2, w3, sw1, sw2, sw3):
    x = hidden_states.reshape(N, H)
    (pos0, pos1, rw0, rw1, te, xbf,
     sw1b, sw2b, sw3b) = _plan(x, gate_w, expert_bias, sw1, sw2, sw3)
    pos0f = pos0.reshape(N)
    pos1f = pos1.reshape(N)
    xs = _dispatch_kernel()(x, pos0f, pos1f)
    ys = _routed_gemm(te.reshape(NT_R), xs, w1, w2, w3)
    sh = _shared_gemm(xbf, sw1b, sw2b, sw3b)
    zs0, zs1 = _combine_kernel()(ys, pos0f, pos1f)
    out = _final(sh, zs0, zs1, rw0, rw1)
    return out.reshape(1, N, H)


# speedup vs baseline: 1.8295x; 1.0175x over previous
import functools

import jax
import jax.numpy as jnp
from jax import lax
from jax.experimental import pallas as pl
from jax.experimental.pallas import tpu as pltpu
from jax.experimental.pallas import tpu_sc as plsc

N, H, I, E = 2048, 1024, 2048, 8
T = 256
NT_R = (2 * N) // T + E
P = NT_R * T
TS = 256
NT = N // TS
NW = 32
TPW = N // NW


def _plan_body(x_ref, gw_ref, bias_ref, pos0_ref, pos1_ref, w0_ref, w1_ref,
               te_ref):
    x = x_ref[...]
    gw = gw_ref[...]
    bias = bias_ref[...]
    logits = lax.dot_general(x, gw, (((1,), (1,)), ((), ())),
                             preferred_element_type=jnp.float32)
    biased = logits + bias
    eiota = lax.broadcasted_iota(jnp.int32, (N, E), 1)
    m0 = jnp.max(biased, axis=1, keepdims=True)
    e0 = jnp.min(jnp.where(biased >= m0, eiota, E), axis=1, keepdims=True)
    oh0 = eiota == e0
    biased2 = jnp.where(oh0, -jnp.inf, biased)
    m1 = jnp.max(biased2, axis=1, keepdims=True)
    e1 = jnp.min(jnp.where(biased2 >= m1, eiota, E), axis=1, keepdims=True)
    oh1 = eiota == e1
    g0 = jnp.sum(jnp.where(oh0, logits, 0.0), axis=1, keepdims=True)
    g1 = jnp.sum(jnp.where(oh1, logits, 0.0), axis=1, keepdims=True)
    mm = jnp.maximum(g0, g1)
    z0 = jnp.exp(g0 - mm)
    z1 = jnp.exp(g1 - mm)
    w0_ref[...] = z0 / (z0 + z1)
    w1_ref[...] = z1 / (z0 + z1)
    both = oh0.astype(jnp.float32) + oh1.astype(jnp.float32)
    r = lax.broadcasted_iota(jnp.int32, (N, N), 0)
    c = lax.broadcasted_iota(jnp.int32, (N, N), 1)
    ltri = (c < r).astype(jnp.bfloat16)
    cex = lax.dot_general(ltri, both.astype(jnp.bfloat16),
                          (((1,), (0,)), ((), ())),
                          preferred_element_type=jnp.float32)
    counts = jnp.sum(both, axis=0, keepdims=True)
    tiles = jnp.floor((counts + (T - 1)) * (1.0 / T))
    er = lax.broadcasted_iota(jnp.int32, (E, E), 0)
    ec = lax.broadcasted_iota(jnp.int32, (E, E), 1)
    strict = (er < ec).astype(jnp.float32)
    toff = lax.dot_general(tiles, strict, (((1,), (0,)), ((), ())),
                           preferred_element_type=jnp.float32)
    off_row = toff * float(T)
    pos0_ref[...] = jnp.sum(jnp.where(oh0, off_row + cex, 0.0), axis=1,
                            keepdims=True).astype(jnp.int32)
    pos1_ref[...] = jnp.sum(jnp.where(oh1, off_row + cex, 0.0), axis=1,
                            keepdims=True).astype(jnp.int32)
    bnd = (toff + tiles).astype(jnp.int32)
    s_row = lax.broadcasted_iota(jnp.int32, (NT_R, E), 0)
    te = jnp.sum((bnd <= s_row).astype(jnp.int32), axis=1, keepdims=True)
    te_ref[...] = te


def _plan(x, gate_w, expert_bias):
    return pl.pallas_call(
        _plan_body,
        out_shape=(
            jax.ShapeDtypeStruct((N, 1), jnp.int32),
            jax.ShapeDtypeStruct((N, 1), jnp.int32),
            jax.ShapeDtypeStruct((N, 1), jnp.float32),
            jax.ShapeDtypeStruct((N, 1), jnp.float32),
            jax.ShapeDtypeStruct((NT_R, 1), jnp.int32),
        ),
    )(x, gate_w, expert_bias.reshape(1, E))


@functools.cache
def _dispatch_kernel():
    @functools.partial(
        pl.kernel,
        out_type=jax.ShapeDtypeStruct((P, H), jnp.float32),
        mesh=plsc.VectorSubcoreMesh(core_axis_name="c", subcore_axis_name="s"),
        scratch_types=[
            pltpu.VMEM((TPW,), jnp.int32),
            pltpu.VMEM((TPW,), jnp.int32),
            pltpu.VMEM((TPW, H), jnp.float32),
            pltpu.SemaphoreType.DMA,
        ],
    )
    def _dispatch(x_hbm, pos0_hbm, pos1_hbm, xs_hbm, idx0_v, idx1_v, rows_v,
                  sem):
        wid = lax.axis_index("s") * 2 + lax.axis_index("c")
        base = wid * TPW
        pltpu.sync_copy(pos0_hbm.at[pl.ds(base, TPW)], idx0_v)
        pltpu.sync_copy(pos1_hbm.at[pl.ds(base, TPW)], idx1_v)
        pltpu.sync_copy(x_hbm.at[pl.ds(base, TPW)], rows_v)
        pltpu.async_copy(rows_v, xs_hbm.at[idx0_v], sem).wait()
        pltpu.async_copy(rows_v, xs_hbm.at[idx1_v], sem).wait()

    return _dispatch


def _gemm_body(te_ref, x_ref, w1_ref, w3_ref, w2_ref, o_ref):
    i = pl.program_id(0)
    te = te_ref[i]

    @pl.when(te < E)
    def _():
        x = x_ref[...]
        w1b = w1_ref[0]
        w3b = w3_ref[0]
        w2b = w2_ref[0]
        h1 = lax.dot_general(x, w1b, (((1,), (1,)), ((), ())),
                             preferred_element_type=jnp.float32)
        h3 = lax.dot_general(x, w3b, (((1,), (1,)), ((), ())),
                             preferred_element_type=jnp.float32)
        g = h1 * jax.nn.sigmoid(h1) * h3
        o_ref[...] = lax.dot_general(g, w2b, (((1,), (1,)), ((), ())),
                                     preferred_element_type=jnp.float32)


def _wsel(te, i):
    return jnp.minimum(te[i], E - 1)


def _routed_gemm(te, xs, w1b, w2b, w3b):
    grid_spec = pltpu.PrefetchScalarGridSpec(
        num_scalar_prefetch=1,
        grid=(NT_R,),
        in_specs=[
            pl.BlockSpec((T, H), lambda i, te: (i, 0)),
            pl.BlockSpec((1, I, H), lambda i, te: (_wsel(te, i), 0, 0)),
            pl.BlockSpec((1, I, H), lambda i, te: (_wsel(te, i), 0, 0)),
            pl.BlockSpec((1, H, I), lambda i, te: (_wsel(te, i), 0, 0)),
        ],
        out_specs=pl.BlockSpec((T, H), lambda i, te: (i, 0)),
    )
    return pl.pallas_call(
        _gemm_body,
        grid_spec=grid_spec,
        out_shape=jax.ShapeDtypeStruct((P, H), jnp.float32),
        compiler_params=pltpu.CompilerParams(
            dimension_semantics=("arbitrary",)),
    )(te, xs, w1b, w3b, w2b)


@functools.cache
def _combine_kernel():
    @functools.partial(
        pl.kernel,
        out_type=(jax.ShapeDtypeStruct((N, H), jnp.float32),
                  jax.ShapeDtypeStruct((N, H), jnp.float32)),
        mesh=plsc.VectorSubcoreMesh(core_axis_name="c", subcore_axis_name="s"),
        scratch_types=[
            pltpu.VMEM((TPW,), jnp.int32),
            pltpu.VMEM((TPW, H), jnp.float32),
            pltpu.SemaphoreType.DMA,
        ],
    )
    def _combine(ys_hbm, pos0_hbm, pos1_hbm, zs0_hbm, zs1_hbm, idx_v, rows_v,
                 sem):
        wid = lax.axis_index("s") * 2 + lax.axis_index("c")
        base = wid * TPW
        pltpu.sync_copy(pos0_hbm.at[pl.ds(base, TPW)], idx_v)
        pltpu.async_copy(ys_hbm.at[idx_v], rows_v, sem).wait()
        pltpu.sync_copy(rows_v, zs0_hbm.at[pl.ds(base, TPW)])
        pltpu.sync_copy(pos1_hbm.at[pl.ds(base, TPW)], idx_v)
        pltpu.async_copy(ys_hbm.at[idx_v], rows_v, sem).wait()
        pltpu.sync_copy(rows_v, zs1_hbm.at[pl.ds(base, TPW)])

    return _combine


def _shared_body(x_ref, w1_ref, w3_ref, w2_ref, o_ref):
    x = x_ref[...]
    h1 = lax.dot_general(x, w1_ref[...], (((1,), (1,)), ((), ())),
                         preferred_element_type=jnp.float32)
    h3 = lax.dot_general(x, w3_ref[...], (((1,), (1,)), ((), ())),
                         preferred_element_type=jnp.float32)
    g = h1 * jax.nn.sigmoid(h1) * h3
    o_ref[...] = lax.dot_general(g, w2_ref[...], (((1,), (1,)), ((), ())),
                                 preferred_element_type=jnp.float32)


def _shared_gemm(x, sw1, sw2, sw3):
    return pl.pallas_call(
        _shared_body,
        grid=(NT,),
        in_specs=[
            pl.BlockSpec((TS, H), lambda i: (i, 0)),
            pl.BlockSpec((I, H), lambda i: (0, 0)),
            pl.BlockSpec((I, H), lambda i: (0, 0)),
            pl.BlockSpec((H, I), lambda i: (0, 0)),
        ],
        out_specs=pl.BlockSpec((TS, H), lambda i: (i, 0)),
        out_shape=jax.ShapeDtypeStruct((N, H), jnp.float32),
        compiler_params=pltpu.CompilerParams(
            dimension_semantics=("arbitrary",)),
    )(x, sw1, sw3, sw2)


def _fin_body(sh_ref, z0_ref, z1_ref, w0_ref, w1_ref, o_ref):
    o_ref[...] = (sh_ref[...] + w0_ref[...] * z0_ref[...]
                  + w1_ref[...] * z1_ref[...])


def _final(sh, zs0, zs1, rw0, rw1):
    return pl.pallas_call(
        _fin_body,
        grid=(NT,),
        in_specs=[
            pl.BlockSpec((TS, H), lambda i: (i, 0)),
            pl.BlockSpec((TS, H), lambda i: (i, 0)),
            pl.BlockSpec((TS, H), lambda i: (i, 0)),
            pl.BlockSpec((TS, 1), lambda i: (i, 0)),
            pl.BlockSpec((TS, 1), lambda i: (i, 0)),
        ],
        out_specs=pl.BlockSpec((TS, H), lambda i: (i, 0)),
        out_shape=jax.ShapeDtypeStruct((N, H), jnp.float32),
        compiler_params=pltpu.CompilerParams(
            dimension_semantics=("parallel",)),
    )(sh, zs0, zs1, rw0, rw1)


def kernel(hidden_states, gate_w, expert_bias, w1, w2, w3, sw1, sw2, sw3):
    x = hidden_states.reshape(N, H)
    pos0, pos1, rw0, rw1, te = _plan(x, gate_w, expert_bias)
    pos0f = pos0.reshape(N)
    pos1f = pos1.reshape(N)
    xs = _dispatch_kernel()(x, pos0f, pos1f)
    ys = _routed_gemm(te.reshape(NT_R), xs, w1, w2, w3)
    sh = _shared_gemm(x, sw1, sw2, sw3)
    zs0, zs1 = _combine_kernel()(ys, pos0f, pos1f)
    out = _final(sh, zs0, zs1, rw0, rw1)
    return out.reshape(1, N, H)
